# baseline (speedup 1.0000x reference)
import jax
import jax.numpy as jnp
from jax.experimental import pallas as pl
from jax.experimental.pallas import tpu as pltpu

B, H, S, D = 8, 16, 2048, 128
Q = 16
NCH = 16
CH = S // NCH


def _copy_scatter_kernel(pos_ref, kc_ref, vc_ref, kv_ref, vv_ref,
                         ok_ref, ov_ref, copy_sem, scat_sem):
    copies = []
    for c in range(NCH):
        for j, (src, dst) in enumerate(((kc_ref, ok_ref), (vc_ref, ov_ref))):
            d = pltpu.make_async_copy(
                src.at[:, :, pl.ds(c * CH, CH), :],
                dst.at[:, :, pl.ds(c * CH, CH), :],
                copy_sem.at[j, c])
            d.start()
            copies.append(d)
    for d in copies:
        d.wait()

    scats = []
    for i in range(Q):
        p = pos_ref[i]
        for j, (val, dst) in enumerate(((kv_ref, ok_ref), (vv_ref, ov_ref))):
            d = pltpu.make_async_copy(
                val.at[:, :, pl.ds(i, 1), :],
                dst.at[:, :, pl.ds(p, 1), :],
                scat_sem.at[j, i])
            d.start()
            scats.append(d)
    for d in scats:
        d.wait()


def kernel(k_cache, v_cache, input_pos, k_val, v_val):
    out_k, out_v = pl.pallas_call(
        _copy_scatter_kernel,
        out_shape=[jax.ShapeDtypeStruct((B, H, S, D), jnp.float32)] * 2,
        in_specs=[
            pl.BlockSpec(memory_space=pltpu.SMEM),
            pl.BlockSpec(memory_space=pl.ANY),
            pl.BlockSpec(memory_space=pl.ANY),
            pl.BlockSpec(memory_space=pl.ANY),
            pl.BlockSpec(memory_space=pl.ANY),
        ],
        out_specs=[pl.BlockSpec(memory_space=pl.ANY)] * 2,
        scratch_shapes=[
            pltpu.SemaphoreType.DMA((2, NCH)),
            pltpu.SemaphoreType.DMA((2, Q)),
        ],
    )(input_pos, k_cache, v_cache, k_val, v_val)
    return (out_k, out_v)


# speedup vs baseline: 43.5355x; 43.5355x over previous
import functools

import jax
import jax.numpy as jnp
from jax.experimental import pallas as pl
from jax.experimental.pallas import tpu as pltpu

B, H, S, D = 8, 16, 2048, 128
Q = 16
BH = B * H


def _copy_scatter_kernel(pos_ref, kc_ref, vc_ref, kv_ref, vv_ref,
                         ok_ref, ov_ref):
    ok_ref[...] = kc_ref[...]
    ov_ref[...] = vc_ref[...]

    p0 = pos_ref[0]
    contig = functools.reduce(
        jnp.logical_and,
        [pos_ref[i] == p0 + i for i in range(1, Q)])

    @pl.when(contig)
    def _():
        ok_ref[0, pl.ds(p0, Q), :] = kv_ref[0]
        ov_ref[0, pl.ds(p0, Q), :] = vv_ref[0]

    @pl.when(jnp.logical_not(contig))
    def _():
        for i in range(Q):
            p = pos_ref[i]
            ok_ref[0, pl.ds(p, 1), :] = kv_ref[0, pl.ds(i, 1), :]
            ov_ref[0, pl.ds(p, 1), :] = vv_ref[0, pl.ds(i, 1), :]


def kernel(k_cache, v_cache, input_pos, k_val, v_val):
    kc = k_cache.reshape(BH, S, D)
    vc = v_cache.reshape(BH, S, D)
    kv = k_val.reshape(BH, Q, D)
    vv = v_val.reshape(BH, Q, D)

    out_k, out_v = pl.pallas_call(
        _copy_scatter_kernel,
        grid=(BH,),
        out_shape=[jax.ShapeDtypeStruct((BH, S, D), jnp.float32)] * 2,
        in_specs=[
            pl.BlockSpec(memory_space=pltpu.SMEM),
            pl.BlockSpec((1, S, D), lambda i: (i, 0, 0)),
            pl.BlockSpec((1, S, D), lambda i: (i, 0, 0)),
            pl.BlockSpec((1, Q, D), lambda i: (i, 0, 0)),
            pl.BlockSpec((1, Q, D), lambda i: (i, 0, 0)),
        ],
        out_specs=[pl.BlockSpec((1, S, D), lambda i: (i, 0, 0))] * 2,
        compiler_params=pltpu.CompilerParams(
            dimension_semantics=("arbitrary",)),
    )(input_pos, kc, vc, kv, vv)
    return (out_k.reshape(B, H, S, D), out_v.reshape(B, H, S, D))
